# baseline (speedup 1.0000x reference)
import functools

import jax
import jax.numpy as jnp
from jax import lax
from jax.experimental import pallas as pl
from jax.experimental.pallas import tpu as pltpu
from jax.experimental.pallas import tpu_sc as plsc

B = 16
P = 512
D = 512
T = 2048
L = 16
FT = 512


def _sc_intervals(dur):
    mesh = plsc.VectorSubcoreMesh(core_axis_name="c", subcore_axis_name="s")

    @functools.partial(
        pl.kernel,
        mesh=mesh,
        compiler_params=pltpu.CompilerParams(needs_layout_passes=False),
        out_type=jax.ShapeDtypeStruct((B, 2 * P), jnp.int32),
        scratch_types=[
            pltpu.VMEM((P,), jnp.int32),
            pltpu.VMEM((2 * P,), jnp.int32),
        ],
    )
    def body(dur_hbm, out_hbm, dur_v, se_v):
        c = lax.axis_index("c")
        s = lax.axis_index("s")
        wid = s * 2 + c

        @pl.when(wid < B)
        def _():
            b = wid
            pltpu.sync_copy(dur_hbm.at[b], dur_v)
            carry = jnp.int32(0)
            for k in range(P // L):
                v = dur_v[pl.ds(k * L, L)]
                ends = plsc.cumsum(v) + carry
                carry = carry + jnp.sum(v)
                se_v[pl.ds(k * L, L)] = ends - v
                se_v[pl.ds(P + k * L, L)] = ends
            pltpu.sync_copy(se_v, out_hbm.at[b])

    return body(dur)


def _tc_body(tot_ref, se_ref, enc_ref, out_ref, ebf_ref):
    b = pl.program_id(0)
    f = pl.program_id(1)
    start_f = f * FT
    tot = tot_ref[b]

    @pl.when(f == 0)
    def _():
        ebf_ref[...] = enc_ref[0].astype(jnp.bfloat16)

    @pl.when(start_f < tot)
    def _():
        st = se_ref[0, 0, :P]
        en = se_ref[0, 0, P:]
        t = lax.broadcasted_iota(jnp.int32, (FT, P), 0) + start_f
        oh = ((t >= st[None, :]) & (t < en[None, :])).astype(jnp.bfloat16)
        out_ref[0] = jnp.dot(oh, ebf_ref[...],
                             preferred_element_type=jnp.float32)

    @pl.when(start_f >= tot)
    def _():
        out_ref[0] = jnp.zeros((FT, D), jnp.float32)


def _tc_expand(totals, se3, enc):
    return pl.pallas_call(
        _tc_body,
        grid=(B, T // FT),
        in_specs=[
            pl.BlockSpec(memory_space=pltpu.SMEM),
            pl.BlockSpec((1, 1, 2 * P), lambda b, f: (b, 0, 0)),
            pl.BlockSpec((1, P, D), lambda b, f: (b, 0, 0)),
        ],
        out_specs=pl.BlockSpec((1, FT, D), lambda b, f: (b, f, 0)),
        out_shape=jax.ShapeDtypeStruct((B, T, D), jnp.float32),
        scratch_shapes=[pltpu.VMEM((P, D), jnp.bfloat16)],
    )(totals, se3, enc)


def kernel(encoder_output, log_durations):
    mask = (log_durations > 0).astype(jnp.int32)
    dur = (jnp.power(2.0, log_durations) + 0.0001).astype(jnp.int32) * mask
    dur = dur.reshape(B, P)
    se = _sc_intervals(dur)
    se3 = se.reshape(B, 1, 2 * P)
    totals = jnp.sum(dur, axis=1)
    return _tc_expand(totals, se3, encoder_output)


# speedup vs baseline: 1.2477x; 1.2477x over previous
import functools

import jax
import jax.numpy as jnp
from jax import lax
from jax.experimental import pallas as pl
from jax.experimental.pallas import tpu as pltpu
from jax.experimental.pallas import tpu_sc as plsc

B = 16
P = 512
D = 512
T = 2048
L = 16
FT = 1024


def _sc_intervals(dur):
    mesh = plsc.VectorSubcoreMesh(core_axis_name="c", subcore_axis_name="s")

    @functools.partial(
        pl.kernel,
        mesh=mesh,
        compiler_params=pltpu.CompilerParams(needs_layout_passes=False),
        out_type=jax.ShapeDtypeStruct((B, 2 * P), jnp.int32),
        scratch_types=[
            pltpu.VMEM((P,), jnp.int32),
            pltpu.VMEM((2 * P,), jnp.int32),
        ],
    )
    def body(dur_hbm, out_hbm, dur_v, se_v):
        c = lax.axis_index("c")
        s = lax.axis_index("s")
        wid = s * 2 + c

        @pl.when(wid < B)
        def _():
            b = wid
            pltpu.sync_copy(dur_hbm.at[b], dur_v)
            carry = jnp.int32(0)
            for k in range(P // L):
                v = dur_v[pl.ds(k * L, L)]
                ends = plsc.cumsum(v) + carry
                carry = carry + jnp.sum(v)
                se_v[pl.ds(k * L, L)] = ends - v
                se_v[pl.ds(P + k * L, L)] = ends
            pltpu.sync_copy(se_v, out_hbm.at[b])

    return body(dur)


def _tc_body(tot_ref, se_ref, enc_ref, out_ref):
    f = pl.program_id(0)
    b = pl.program_id(1)
    start_f = f * FT
    tot = tot_ref[b]

    @pl.when(start_f < tot)
    def _():
        st = se_ref[0, 0, :P]
        en = se_ref[0, 0, P:]
        t = lax.broadcasted_iota(jnp.int32, (FT, P), 0) + start_f
        oh = ((t >= st[None, :]) & (t < en[None, :])).astype(jnp.bfloat16)
        out_ref[0] = jnp.dot(oh, enc_ref[0].astype(jnp.bfloat16),
                             preferred_element_type=jnp.float32)

    @pl.when(start_f >= tot)
    def _():
        out_ref[0] = jnp.zeros((FT, D), jnp.float32)


def _tc_expand(totals, se3, enc):
    return pl.pallas_call(
        _tc_body,
        grid=(T // FT, B),
        in_specs=[
            pl.BlockSpec(memory_space=pltpu.SMEM),
            pl.BlockSpec((1, 1, 2 * P), lambda f, b: (b, 0, 0)),
            pl.BlockSpec((1, P, D), lambda f, b: (b, 0, 0)),
        ],
        out_specs=pl.BlockSpec((1, FT, D), lambda f, b: (b, f, 0)),
        out_shape=jax.ShapeDtypeStruct((B, T, D), jnp.float32),
    )(totals, se3, enc)


def kernel(encoder_output, log_durations):
    mask = (log_durations > 0).astype(jnp.int32)
    dur = (jnp.power(2.0, log_durations) + 0.0001).astype(jnp.int32) * mask
    dur = dur.reshape(B, P)
    se = _sc_intervals(dur)
    se3 = se.reshape(B, 1, 2 * P)
    totals = jnp.sum(dur, axis=1)
    return _tc_expand(totals, se3, encoder_output)


# speedup vs baseline: 1.2617x; 1.0112x over previous
import functools

import jax
import jax.numpy as jnp
from jax import lax
from jax.experimental import pallas as pl
from jax.experimental.pallas import tpu as pltpu
from jax.experimental.pallas import tpu_sc as plsc

B = 16
P = 512
D = 512
T = 2048
L = 16
NTILES = 32
FRAMES_PER_TILE = B * T // NTILES
CHUNK = 64
NCHUNK = FRAMES_PER_TILE // CHUNK
HALF_T = T // 2


def _sc_length_regulate(enc_flat, dur, zero_blk):
    mesh = plsc.VectorSubcoreMesh(core_axis_name="c", subcore_axis_name="s")

    @functools.partial(
        pl.kernel,
        mesh=mesh,
        compiler_params=pltpu.CompilerParams(needs_layout_passes=False),
        out_type=jax.ShapeDtypeStruct((B * T, D), jnp.float32),
        scratch_types=[
            pltpu.VMEM((P,), jnp.int32),
            pltpu.VMEM((FRAMES_PER_TILE,), jnp.int32),
            pltpu.VMEM((NCHUNK, CHUNK), jnp.int32),
            pltpu.VMEM((CHUNK, D), jnp.float32),
            pltpu.VMEM((CHUNK, D), jnp.float32),
            pltpu.VMEM((CHUNK, D), jnp.float32),
            pltpu.SemaphoreType.DMA,
            pltpu.SemaphoreType.DMA,
            pltpu.SemaphoreType.DMA,
            pltpu.SemaphoreType.DMA,
        ],
    )
    def body(enc_hbm, dur_hbm, zero_hbm, out_hbm,
             dur_v, map_v, idx_v, buf_a, buf_b, zero_v,
             gsem_a, gsem_b, wsem_a, wsem_b):
        c = lax.axis_index("c")
        s = lax.axis_index("s")
        wid = s * 2 + c
        b = wid % B
        base = (wid // B) * HALF_T
        row0 = b * T + base

        pltpu.sync_copy(dur_hbm.at[b], dur_v)
        pltpu.sync_copy(zero_hbm, zero_v)

        neg1 = jnp.full((L,), -1, jnp.int32)
        for k in range(FRAMES_PER_TILE // L):
            map_v[pl.ds(k * L, L)] = neg1

        lane = lax.iota(jnp.int32, L)
        carry = jnp.int32(0)
        acc = neg1
        for k in range(P // L):
            v = dur_v[pl.ds(k * L, L)]
            ends = plsc.cumsum(v) + carry
            carry = carry + jnp.sum(v)
            starts = ends - v
            pid = lane + (k * L)
            loc = starts - base
            m = (v > 0) & (loc >= 0) & (loc < FRAMES_PER_TILE)
            plsc.store_scatter(map_v, [loc], pid, mask=m)
            acc = jnp.maximum(acc, jnp.where((v > 0) & (starts < base), pid, -1))
        total = carry
        pc = jnp.max(acc)

        rowbase = b * P
        vecs_per_chunk = CHUNK // L
        for k in range(FRAMES_PER_TILE // L):
            v = map_v[pl.ds(k * L, L)]
            ph = jnp.maximum(plsc.cummax(v), pc)
            pc = jnp.max(ph)
            idx_v[k // vecs_per_chunk,
                  pl.ds((k % vecs_per_chunk) * L, L)] = jnp.clip(ph, 0, P - 1) + rowbase

        bufs = (buf_a, buf_b)
        gsems = (gsem_a, gsem_b)
        wsems = (wsem_a, wsem_b)

        def n_of(j):
            return jnp.clip(total - (base + j * CHUNK), 0, CHUNK)

        def dst_of(j):
            return out_hbm.at[pl.ds(row0 + j * CHUNK, CHUNK)]

        def fire_gather(j, q):
            @pl.when(n_of(j) > 0)
            def _():
                pltpu.async_copy(enc_hbm.at[idx_v.at[j]], bufs[q], gsems[q])

        fire_gather(0, 0)
        fire_gather(1, 1)
        for j in range(NCHUNK):
            q = j & 1
            n = n_of(j)

            @pl.when(n > 0)
            def _(j=j, q=q, n=n):
                pltpu.make_async_copy(
                    enc_hbm.at[idx_v.at[j]], bufs[q], gsems[q]).wait()

                @pl.when(n < CHUNK)
                def _():
                    def zero_row(r, carry_):
                        for cc in range(D // L):
                            bufs[q][r, pl.ds(cc * L, L)] = jnp.zeros(
                                (L,), jnp.float32)
                        return carry_
                    lax.fori_loop(n, CHUNK, zero_row, 0)

                pltpu.async_copy(bufs[q], dst_of(j), wsems[q])

            @pl.when(n <= 0)
            def _(j=j, q=q):
                pltpu.async_copy(zero_v, dst_of(j), wsems[q])

            if j + 2 < NCHUNK:
                pltpu.make_async_copy(bufs[q], dst_of(j), wsems[q]).wait()
                fire_gather(j + 2, q)

        pltpu.make_async_copy(buf_a, dst_of(NCHUNK - 2), wsem_a).wait()
        pltpu.make_async_copy(buf_b, dst_of(NCHUNK - 1), wsem_b).wait()

    return body(enc_flat, dur, zero_blk)


def kernel(encoder_output, log_durations):
    mask = (log_durations > 0).astype(jnp.int32)
    dur = (jnp.power(2.0, log_durations) + 0.0001).astype(jnp.int32) * mask
    dur = dur.reshape(B, P)
    enc_flat = encoder_output.reshape(B * P, D)
    zero_blk = jnp.zeros((CHUNK, D), jnp.float32)
    out = _sc_length_regulate(enc_flat, dur, zero_blk)
    return out.reshape(B, T, D)


# speedup vs baseline: 1.2919x; 1.0239x over previous
import functools

import jax
import jax.numpy as jnp
from jax import lax
from jax.experimental import pallas as pl
from jax.experimental.pallas import tpu as pltpu
from jax.experimental.pallas import tpu_sc as plsc

B = 16
P = 512
D = 512
T = 2048
L = 16
NTILES = 32
FRAMES_PER_TILE = B * T // NTILES
CHUNK = 64
NCHUNK = FRAMES_PER_TILE // CHUNK
HALF_T = T // 2


def _sc_length_regulate(enc_flat, dur, zero_blk):
    mesh = plsc.VectorSubcoreMesh(core_axis_name="c", subcore_axis_name="s")

    @functools.partial(
        pl.kernel,
        mesh=mesh,
        compiler_params=pltpu.CompilerParams(needs_layout_passes=False),
        out_type=jax.ShapeDtypeStruct((B * T, D), jnp.float32),
        scratch_types=[
            pltpu.VMEM((P,), jnp.int32),
            pltpu.VMEM((FRAMES_PER_TILE,), jnp.int32),
            pltpu.VMEM((NCHUNK, CHUNK), jnp.int32),
            pltpu.VMEM((CHUNK, D), jnp.float32),
            pltpu.VMEM((CHUNK, D), jnp.float32),
            pltpu.VMEM((CHUNK, D), jnp.float32),
            pltpu.VMEM((CHUNK // 2, D), jnp.float32),
            pltpu.SemaphoreType.DMA,
            pltpu.SemaphoreType.DMA,
            pltpu.SemaphoreType.DMA,
            pltpu.SemaphoreType.DMA,
            pltpu.SemaphoreType.DMA,
            pltpu.SemaphoreType.DMA,
            pltpu.SemaphoreType.DMA,
        ],
    )
    def body(enc_hbm, dur_hbm, zero_hbm, out_hbm,
             dur_v, map_v, idx_v, buf_0, buf_1, buf_2, zero_v,
             gsem_0, gsem_1, gsem_2, wsem_0, wsem_1, wsem_2, zsem):
        c = lax.axis_index("c")
        s = lax.axis_index("s")
        wid = s * 2 + c
        b = wid % B
        base = (wid // B) * HALF_T
        row0 = b * T + base

        pltpu.sync_copy(dur_hbm.at[b], dur_v)
        pltpu.sync_copy(zero_hbm, zero_v)

        neg1 = jnp.full((L,), -1, jnp.int32)
        for k in range(FRAMES_PER_TILE // L):
            map_v[pl.ds(k * L, L)] = neg1

        lane = lax.iota(jnp.int32, L)
        carry = jnp.int32(0)
        acc = neg1
        for k in range(P // L):
            v = dur_v[pl.ds(k * L, L)]
            ends = plsc.cumsum(v) + carry
            carry = carry + jnp.sum(v)
            starts = ends - v
            pid = lane + (k * L)
            loc = starts - base
            m = (v > 0) & (loc >= 0) & (loc < FRAMES_PER_TILE)
            plsc.store_scatter(map_v, [loc], pid, mask=m)
            acc = jnp.maximum(acc, jnp.where((v > 0) & (starts < base), pid, -1))
        total = carry
        pc = jnp.max(acc)

        rowbase = b * P
        vecs_per_chunk = CHUNK // L
        for k in range(FRAMES_PER_TILE // L):
            v = map_v[pl.ds(k * L, L)]
            ph = jnp.maximum(plsc.cummax(v), pc)
            pc = jnp.max(ph)
            idx_v[k // vecs_per_chunk,
                  pl.ds((k % vecs_per_chunk) * L, L)] = jnp.clip(ph, 0, P - 1) + rowbase

        bufs = (buf_0, buf_1, buf_2)
        gsems = (gsem_0, gsem_1, gsem_2)
        wsems = (wsem_0, wsem_1, wsem_2)
        HC = CHUNK // 2

        def n_of(j):
            return jnp.clip(total - (base + j * CHUNK), 0, CHUNK)

        def dst_of(j):
            return out_hbm.at[pl.ds(row0 + j * CHUNK, CHUNK)]

        def zdst_of(j, h):
            return out_hbm.at[pl.ds(row0 + j * CHUNK + h * HC, HC)]

        def fire_gather(j):
            @pl.when(n_of(j) > 0)
            def _():
                pltpu.async_copy(enc_hbm.at[idx_v.at[j]],
                                 bufs[j % 3], gsems[j % 3])

        fire_gather(0)
        for j in range(NCHUNK):
            q = j % 3
            n = n_of(j)

            if j >= 2:
                @pl.when(n_of(j - 2) > 0)
                def _(j=j):
                    pltpu.make_async_copy(
                        bufs[(j - 2) % 3], dst_of(j - 2),
                        wsems[(j - 2) % 3]).wait()
            if j + 1 < NCHUNK:
                fire_gather(j + 1)

            @pl.when(n > 0)
            def _(j=j, q=q, n=n):
                pltpu.make_async_copy(
                    enc_hbm.at[idx_v.at[j]], bufs[q], gsems[q]).wait()

                @pl.when(n < CHUNK)
                def _():
                    def zero_row(r, carry_):
                        for cc in range(D // L):
                            bufs[q][r, pl.ds(cc * L, L)] = jnp.zeros(
                                (L,), jnp.float32)
                        return carry_
                    lax.fori_loop(n, CHUNK, zero_row, 0)

                pltpu.async_copy(bufs[q], dst_of(j), wsems[q])

            @pl.when(n <= 0)
            def _(j=j):
                pltpu.async_copy(zero_v, zdst_of(j, 0), zsem)
                pltpu.async_copy(zero_v, zdst_of(j, 1), zsem)

        for j in (NCHUNK - 2, NCHUNK - 1):
            @pl.when(n_of(j) > 0)
            def _(j=j):
                pltpu.make_async_copy(bufs[j % 3], dst_of(j),
                                      wsems[j % 3]).wait()
        for j in range(NCHUNK):
            @pl.when(n_of(j) <= 0)
            def _(j=j):
                pltpu.make_async_copy(zero_v, zdst_of(j, 0), zsem).wait()
                pltpu.make_async_copy(zero_v, zdst_of(j, 1), zsem).wait()

    return body(enc_flat, dur, zero_blk)


def kernel(encoder_output, log_durations):
    mask = (log_durations > 0).astype(jnp.int32)
    dur = (jnp.power(2.0, log_durations) + 0.0001).astype(jnp.int32) * mask
    dur = dur.reshape(B, P)
    enc_flat = encoder_output.reshape(B * P, D)
    zero_blk = jnp.zeros((CHUNK // 2, D), jnp.float32)
    out = _sc_length_regulate(enc_flat, dur, zero_blk)
    return out.reshape(B, T, D)
